# speedup vs baseline: 27.0138x; 27.0138x over previous
import functools

import jax
import jax.numpy as jnp
from jax import lax
from jax.experimental import pallas as pl
from jax.experimental.pallas import tpu as pltpu
from jax.experimental.pallas import tpu_sc as plsc

DIM = 128
BATCH = 16384
LANES = 128
NUM_CORES = 2
NUM_SUBCORES = 16
NW = NUM_CORES * NUM_SUBCORES
B_PER_W = BATCH // NW
CHUNKS = B_PER_W // LANES

_mesh = plsc.VectorSubcoreMesh(core_axis_name="c", subcore_axis_name="s")


@functools.partial(
    pl.kernel,
    mesh=_mesh,
    out_type=[
        jax.ShapeDtypeStruct((BATCH, DIM), jnp.float32),
        jax.ShapeDtypeStruct((BATCH, DIM), jnp.float32),
    ],
    scratch_types=[
        pltpu.VMEM((CHUNKS, LANES), jnp.int32),
        pltpu.VMEM((B_PER_W, DIM), jnp.float32),
        pltpu.SemaphoreType.DMA,
    ],
)
def _sc_gather(doc_idx, neg_idx, table, dc_out, nc_out, idx_v, rows_v, sem):
    wid = lax.axis_index("s") * NUM_CORES + lax.axis_index("c")
    base_row = wid * CHUNKS
    base_out = wid * B_PER_W
    for idx_hbm, out_hbm in ((doc_idx, dc_out), (neg_idx, nc_out)):
        pltpu.sync_copy(idx_hbm.at[pl.ds(base_row, CHUNKS)], idx_v)
        copies = [
            pltpu.async_copy(
                table.at[idx_v.at[j]],
                rows_v.at[pl.ds(j * LANES, LANES)],
                sem,
            )
            for j in range(CHUNKS)
        ]
        for cp in copies:
            cp.wait()
        pltpu.sync_copy(rows_v, out_hbm.at[pl.ds(base_out, B_PER_W)])


def kernel(doc_center_ids, neg_center_ids, center_vecs):
    doc2 = doc_center_ids.reshape(BATCH // LANES, LANES)
    neg2 = neg_center_ids.reshape(BATCH // LANES, LANES)
    dc_emb, nc_emb = _sc_gather(doc2, neg2, center_vecs)
    return dc_emb, nc_emb
